# Initial kernel scaffold; baseline (speedup 1.0000x reference)
import jax
import jax.numpy as jnp
from jax.experimental import pallas as pl


def kernel(x, emb_table, W_ih, W_hh, b_ih, b_hh, W_out, b_out):
    raise NotImplementedError("write your pallas kernel here")


# speedup vs baseline: 5.8841x; 5.8841x over previous
import functools

import jax
import jax.numpy as jnp
from jax import lax
from jax.experimental import pallas as pl
from jax.experimental.pallas import tpu as pltpu
import jax.experimental.pallas.tpu_sc as plsc

V = 1000
VP = 1024
E = 256
H = 1024
G = 4 * H
C = 2
B = 4
T = 2048
N = B * T


_PROJ_BM = 128


def _proj_body(emb_ref, w_ref, b_ref, out_ref):
    p = jnp.dot(emb_ref[...], w_ref[...], preferred_element_type=jnp.float32)
    par = lax.broadcasted_iota(jnp.int32, (_PROJ_BM, 1), 0) % 2
    even = p[:, :G] + b_ref[0:1, :]
    odd = p[:, G:] + b_ref[1:2, :]
    out_ref[...] = jnp.where(par == 0, even, odd)


def _proj_table(emb_p, wih_cat, bias2):
    return pl.pallas_call(
        _proj_body,
        grid=(VP // _PROJ_BM,),
        in_specs=[
            pl.BlockSpec((_PROJ_BM, E), lambda i: (i, 0)),
            pl.BlockSpec((E, C * G), lambda i: (0, 0)),
            pl.BlockSpec((C, G), lambda i: (0, 0)),
        ],
        out_specs=pl.BlockSpec((_PROJ_BM, G), lambda i: (i, 0)),
        out_shape=jax.ShapeDtypeStruct((VP, G), jnp.float32),
    )(emb_p, wih_cat, bias2)


_NW = 32
_ROWS_PER_W = N // _NW
_CH = 16


def _sc_gather(table, idx):
    mesh = plsc.VectorSubcoreMesh(core_axis_name="c", subcore_axis_name="s")

    @functools.partial(
        pl.kernel,
        out_type=jax.ShapeDtypeStruct((N, G), jnp.float32),
        mesh=mesh,
        scratch_types=[
            pltpu.VMEM((_CH,), jnp.int32),
            pltpu.VMEM((_CH, G), jnp.float32),
            pltpu.SemaphoreType.DMA,
        ],
    )
    def gather_k(table_hbm, idx_hbm, out_hbm, idx_v, rows_v, sem):
        wid = lax.axis_index("s") * 2 + lax.axis_index("c")
        base = wid * _ROWS_PER_W
        for ci in range(_ROWS_PER_W // _CH):
            start = base + ci * _CH
            pltpu.sync_copy(idx_hbm.at[pl.ds(start, _CH)], idx_v)
            pltpu.async_copy(table_hbm.at[idx_v], rows_v, sem).wait()
            pltpu.sync_copy(rows_v, out_hbm.at[pl.ds(start, _CH)])

    return gather_k(table, idx)


_TCH = 64


def _rec_body(cell_sm, ih_ref, whh_ref, hs_ref, h_ref, c_ref):
    blk = pl.program_id(0)

    @pl.when(blk == 0)
    def _():
        h_ref[...] = jnp.zeros((B, H), jnp.float32)
        c_ref[...] = jnp.zeros((B, H), jnp.float32)

    def step(tt, carry):
        h, c = carry
        t = blk * _TCH + tt
        ih = ih_ref[tt]
        hh = jnp.dot(h, whh_ref[...],
                     preferred_element_type=jnp.float32)
        rows = []
        for b in range(B):
            m = cell_sm[b, t]
            rows.append(ih[b:b + 1, :] + hh[b:b + 1, :G]
                        + m * (hh[b:b + 1, G:] - hh[b:b + 1, :G]))
        g = jnp.concatenate(rows, axis=0)
        i = jax.nn.sigmoid(g[:, 0 * H:1 * H])
        f = jax.nn.sigmoid(g[:, 1 * H:2 * H])
        gg = jnp.tanh(g[:, 2 * H:3 * H])
        o = jax.nn.sigmoid(g[:, 3 * H:4 * H])
        c2 = f * c + i * gg
        h2 = o * jnp.tanh(c2)
        hs_ref[tt] = h2
        return (h2, c2)

    h, c = lax.fori_loop(0, _TCH, step, (h_ref[...], c_ref[...]))
    h_ref[...] = h
    c_ref[...] = c


def _recurrence(cellf, ih3, whh_cat):
    return pl.pallas_call(
        _rec_body,
        grid=(T // _TCH,),
        in_specs=[
            pl.BlockSpec(memory_space=pltpu.SMEM),
            pl.BlockSpec((_TCH, B, G), lambda i: (i, 0, 0)),
            pl.BlockSpec((H, C * G), lambda i: (0, 0)),
        ],
        out_specs=pl.BlockSpec((_TCH, B, H), lambda i: (i, 0, 0)),
        out_shape=jax.ShapeDtypeStruct((T, B, H), jnp.float32),
        scratch_shapes=[
            pltpu.VMEM((B, H), jnp.float32),
            pltpu.VMEM((B, H), jnp.float32),
        ],
        compiler_params=pltpu.CompilerParams(
            vmem_limit_bytes=112 * 1024 * 1024,
        ),
    )(cellf, ih3, whh_cat)


_DEC_BM = 512


def _dec_body(h_ref, w_ref, b_ref, out_ref):
    d = jnp.dot(h_ref[...], w_ref[...], preferred_element_type=jnp.float32)
    d = d + b_ref[...]
    mx = jnp.max(d, axis=1, keepdims=True)
    z = d - mx
    s = jnp.sum(jnp.exp(z), axis=1, keepdims=True)
    out_ref[...] = z - jnp.log(s)


def _decoder(hs2, w_p, b_p):
    return pl.pallas_call(
        _dec_body,
        grid=(N // _DEC_BM,),
        in_specs=[
            pl.BlockSpec((_DEC_BM, H), lambda i: (i, 0)),
            pl.BlockSpec((H, VP), lambda i: (0, 0)),
            pl.BlockSpec((1, VP), lambda i: (0, 0)),
        ],
        out_specs=pl.BlockSpec((_DEC_BM, VP), lambda i: (i, 0)),
        out_shape=jax.ShapeDtypeStruct((N, VP), jnp.float32),
    )(hs2, w_p, b_p)


def kernel(x, emb_table, W_ih, W_hh, b_ih, b_hh, W_out, b_out):
    wih_cat = jnp.concatenate([W_ih[0].T, W_ih[1].T], axis=1)
    whh_cat = jnp.concatenate([W_hh[0].T, W_hh[1].T], axis=1)
    bias2 = b_ih + b_hh
    emb_p = jnp.pad(emb_table, ((0, VP - V), (0, 0)))

    proj = _proj_table(emb_p, wih_cat, bias2)

    idx = x.T.reshape(-1).astype(jnp.int32)
    ih = _sc_gather(proj, idx)

    cellf = (x % 2).astype(jnp.float32)
    hs = _recurrence(cellf, ih.reshape(T, B, G), whh_cat)

    hs2 = hs.transpose(1, 0, 2).reshape(N, H)
    w_p = jnp.pad(W_out, ((0, VP - V), (0, 0))).T
    b_p = jnp.pad(b_out, (0, VP - V), constant_values=-1e30).reshape(1, VP)
    logits = _decoder(hs2, w_p, b_p)
    return logits[:, :V]


# speedup vs baseline: 5.9000x; 1.0027x over previous
import functools

import jax
import jax.numpy as jnp
from jax import lax
from jax.experimental import pallas as pl
from jax.experimental.pallas import tpu as pltpu
import jax.experimental.pallas.tpu_sc as plsc

V = 1000
VP = 1024
E = 256
H = 1024
G = 4 * H
C = 2
B = 4
T = 2048
N = B * T


_PROJ_BM = 128


def _proj_body(emb_ref, w_ref, b_ref, out_ref):
    p = jnp.dot(emb_ref[...], w_ref[...], preferred_element_type=jnp.float32)
    par = lax.broadcasted_iota(jnp.int32, (_PROJ_BM, 1), 0) % 2
    even = p[:, :G] + b_ref[0:1, :]
    odd = p[:, G:] + b_ref[1:2, :]
    out_ref[...] = jnp.where(par == 0, even, odd)


def _proj_table(emb_p, wih_cat, bias2):
    return pl.pallas_call(
        _proj_body,
        grid=(VP // _PROJ_BM,),
        in_specs=[
            pl.BlockSpec((_PROJ_BM, E), lambda i: (i, 0)),
            pl.BlockSpec((E, C * G), lambda i: (0, 0)),
            pl.BlockSpec((C, G), lambda i: (0, 0)),
        ],
        out_specs=pl.BlockSpec((_PROJ_BM, G), lambda i: (i, 0)),
        out_shape=jax.ShapeDtypeStruct((VP, G), jnp.float32),
    )(emb_p, wih_cat, bias2)


_NW = 32
_ROWS_PER_W = N // _NW
_CH = 16


def _sc_gather(table, idx):
    mesh = plsc.VectorSubcoreMesh(core_axis_name="c", subcore_axis_name="s")

    @functools.partial(
        pl.kernel,
        out_type=jax.ShapeDtypeStruct((N, G), jnp.float32),
        mesh=mesh,
        scratch_types=[
            pltpu.VMEM((_CH,), jnp.int32),
            pltpu.VMEM((_CH, G), jnp.float32),
            pltpu.SemaphoreType.DMA,
        ],
    )
    def gather_k(table_hbm, idx_hbm, out_hbm, idx_v, rows_v, sem):
        wid = lax.axis_index("s") * 2 + lax.axis_index("c")
        base = wid * _ROWS_PER_W
        for ci in range(_ROWS_PER_W // _CH):
            start = base + ci * _CH
            pltpu.sync_copy(idx_hbm.at[pl.ds(start, _CH)], idx_v)
            pltpu.async_copy(table_hbm.at[idx_v], rows_v, sem).wait()
            pltpu.sync_copy(rows_v, out_hbm.at[pl.ds(start, _CH)])

    return gather_k(table, idx)


_TCH = 64


def _rec_body(cell_sm, ih_ref, whh_ref, hs_ref, h_ref, c_ref):
    blk = pl.program_id(0)

    @pl.when(blk == 0)
    def _():
        h_ref[...] = jnp.zeros((B, H), jnp.float32)
        c_ref[...] = jnp.zeros((B, H), jnp.float32)

    def step(tt, carry):
        h, c = carry
        t = blk * _TCH + tt
        ih = ih_ref[tt]
        hh = jnp.dot(h.astype(jnp.bfloat16), whh_ref[...],
                     preferred_element_type=jnp.float32)
        rows = []
        for b in range(B):
            m = cell_sm[b, t]
            rows.append(ih[b:b + 1, :] + hh[b:b + 1, :G]
                        + m * (hh[b:b + 1, G:] - hh[b:b + 1, :G]))
        g = jnp.concatenate(rows, axis=0)
        i = jax.nn.sigmoid(g[:, 0 * H:1 * H])
        f = jax.nn.sigmoid(g[:, 1 * H:2 * H])
        gg = jnp.tanh(g[:, 2 * H:3 * H])
        o = jax.nn.sigmoid(g[:, 3 * H:4 * H])
        c2 = f * c + i * gg
        h2 = o * jnp.tanh(c2)
        hs_ref[tt] = h2
        return (h2, c2)

    h, c = lax.fori_loop(0, _TCH, step, (h_ref[...], c_ref[...]))
    h_ref[...] = h
    c_ref[...] = c


def _recurrence(cellf, ih3, whh_cat):
    return pl.pallas_call(
        _rec_body,
        grid=(T // _TCH,),
        in_specs=[
            pl.BlockSpec(memory_space=pltpu.SMEM),
            pl.BlockSpec((_TCH, B, G), lambda i: (i, 0, 0)),
            pl.BlockSpec((H, C * G), lambda i: (0, 0)),
        ],
        out_specs=pl.BlockSpec((_TCH, B, H), lambda i: (i, 0, 0)),
        out_shape=jax.ShapeDtypeStruct((T, B, H), jnp.float32),
        scratch_shapes=[
            pltpu.VMEM((B, H), jnp.float32),
            pltpu.VMEM((B, H), jnp.float32),
        ],
        compiler_params=pltpu.CompilerParams(
            vmem_limit_bytes=112 * 1024 * 1024,
        ),
    )(cellf, ih3, whh_cat)


_DEC_BM = 512


def _dec_body(h_ref, w_ref, b_ref, out_ref):
    d = jnp.dot(h_ref[...], w_ref[...], preferred_element_type=jnp.float32)
    d = d + b_ref[...]
    mx = jnp.max(d, axis=1, keepdims=True)
    z = d - mx
    s = jnp.sum(jnp.exp(z), axis=1, keepdims=True)
    out_ref[...] = z - jnp.log(s)


def _decoder(hs2, w_p, b_p):
    return pl.pallas_call(
        _dec_body,
        grid=(N // _DEC_BM,),
        in_specs=[
            pl.BlockSpec((_DEC_BM, H), lambda i: (i, 0)),
            pl.BlockSpec((H, VP), lambda i: (0, 0)),
            pl.BlockSpec((1, VP), lambda i: (0, 0)),
        ],
        out_specs=pl.BlockSpec((_DEC_BM, VP), lambda i: (i, 0)),
        out_shape=jax.ShapeDtypeStruct((N, VP), jnp.float32),
    )(hs2, w_p, b_p)


def kernel(x, emb_table, W_ih, W_hh, b_ih, b_hh, W_out, b_out):
    wih_cat = jnp.concatenate([W_ih[0].T, W_ih[1].T], axis=1)
    whh_cat = jnp.concatenate([W_hh[0].T, W_hh[1].T], axis=1)
    bias2 = b_ih + b_hh
    emb_p = jnp.pad(emb_table, ((0, VP - V), (0, 0)))

    proj = _proj_table(emb_p, wih_cat, bias2)

    idx = x.T.reshape(-1).astype(jnp.int32)
    ih = _sc_gather(proj, idx)

    cellf = (x % 2).astype(jnp.float32)
    hs = _recurrence(cellf, ih.reshape(T, B, G),
                     whh_cat.astype(jnp.bfloat16))

    hs2 = hs.transpose(1, 0, 2).reshape(N, H)
    w_p = jnp.pad(W_out, ((0, VP - V), (0, 0))).T
    b_p = jnp.pad(b_out, (0, VP - V), constant_values=-1e30).reshape(1, VP)
    logits = _decoder(hs2, w_p, b_p)
    return logits[:, :V]


# speedup vs baseline: 6.0479x; 1.0251x over previous
import functools

import jax
import jax.numpy as jnp
from jax import lax
from jax.experimental import pallas as pl
from jax.experimental.pallas import tpu as pltpu
import jax.experimental.pallas.tpu_sc as plsc

V = 1000
VP = 1024
E = 256
H = 1024
G = 4 * H
C = 2
B = 4
T = 2048
N = B * T


_PROJ_BM = 128


def _proj_body(emb_ref, w_ref, b_ref, out_ref):
    p = jnp.dot(emb_ref[...], w_ref[...], preferred_element_type=jnp.float32)
    par = lax.broadcasted_iota(jnp.int32, (_PROJ_BM, 1), 0) % 2
    even = p[:, :G] + b_ref[0:1, :]
    odd = p[:, G:] + b_ref[1:2, :]
    out_ref[...] = jnp.where(par == 0, even, odd)


def _proj_table(emb_p, wih_cat, bias2):
    return pl.pallas_call(
        _proj_body,
        grid=(VP // _PROJ_BM,),
        in_specs=[
            pl.BlockSpec((_PROJ_BM, E), lambda i: (i, 0)),
            pl.BlockSpec((E, C * G), lambda i: (0, 0)),
            pl.BlockSpec((C, G), lambda i: (0, 0)),
        ],
        out_specs=pl.BlockSpec((_PROJ_BM, G), lambda i: (i, 0)),
        out_shape=jax.ShapeDtypeStruct((VP, G), jnp.float32),
    )(emb_p, wih_cat, bias2)


_NW = 32
_ROWS_PER_W = N // _NW
_CH = 16


def _sc_gather(table, idx):
    mesh = plsc.VectorSubcoreMesh(core_axis_name="c", subcore_axis_name="s")

    @functools.partial(
        pl.kernel,
        out_type=jax.ShapeDtypeStruct((N, G), jnp.float32),
        mesh=mesh,
        scratch_types=[
            pltpu.VMEM((_CH,), jnp.int32),
            pltpu.VMEM((_CH, G), jnp.float32),
            pltpu.SemaphoreType.DMA,
        ],
    )
    def gather_k(table_hbm, idx_hbm, out_hbm, idx_v, rows_v, sem):
        wid = lax.axis_index("s") * 2 + lax.axis_index("c")
        base = wid * _ROWS_PER_W
        for ci in range(_ROWS_PER_W // _CH):
            start = base + ci * _CH
            pltpu.sync_copy(idx_hbm.at[pl.ds(start, _CH)], idx_v)
            pltpu.async_copy(table_hbm.at[idx_v], rows_v, sem).wait()
            pltpu.sync_copy(rows_v, out_hbm.at[pl.ds(start, _CH)])

    return gather_k(table, idx)


_TCH = 64


def _rec_body(cell_sm, ih_ref, whh_ref, hs_ref, h_ref, c_ref):
    blk = pl.program_id(0)

    @pl.when(blk == 0)
    def _():
        h_ref[...] = jnp.zeros((B, H), jnp.float32)
        c_ref[...] = jnp.zeros((B, H), jnp.float32)

    def one_step(tt, h, c):
        t = blk * _TCH + tt
        ih = ih_ref[tt]
        hh = jnp.dot(h.astype(jnp.bfloat16), whh_ref[...],
                     preferred_element_type=jnp.float32)
        rows = []
        for b in range(B):
            m = cell_sm[b, t]
            rows.append(ih[b:b + 1, :] + hh[b:b + 1, :G]
                        + m * (hh[b:b + 1, G:] - hh[b:b + 1, :G]))
        g = jnp.concatenate(rows, axis=0)
        i = jax.nn.sigmoid(g[:, 0 * H:1 * H])
        f = jax.nn.sigmoid(g[:, 1 * H:2 * H])
        gg = jnp.tanh(g[:, 2 * H:3 * H])
        o = jax.nn.sigmoid(g[:, 3 * H:4 * H])
        c2 = f * c + i * gg
        h2 = o * jnp.tanh(c2)
        hs_ref[tt] = h2.astype(jnp.bfloat16)
        return h2, c2

    def step(u, carry):
        h, c = carry
        h, c = one_step(2 * u, h, c)
        h, c = one_step(2 * u + 1, h, c)
        return (h, c)

    h, c = lax.fori_loop(0, _TCH // 2, step, (h_ref[...], c_ref[...]))
    h_ref[...] = h
    c_ref[...] = c


def _recurrence(cellf, ih3, whh_b):
    return pl.pallas_call(
        _rec_body,
        grid=(T // _TCH,),
        in_specs=[
            pl.BlockSpec(memory_space=pltpu.SMEM),
            pl.BlockSpec((_TCH, B, G), lambda i: (i, 0, 0)),
            pl.BlockSpec((H, C * G), lambda i: (0, 0)),
        ],
        out_specs=pl.BlockSpec((_TCH, B, H), lambda i: (i, 0, 0)),
        out_shape=jax.ShapeDtypeStruct((T, B, H), jnp.bfloat16),
        scratch_shapes=[
            pltpu.VMEM((B, H), jnp.float32),
            pltpu.VMEM((B, H), jnp.float32),
        ],
        compiler_params=pltpu.CompilerParams(
            vmem_limit_bytes=112 * 1024 * 1024,
        ),
    )(cellf, ih3, whh_b)


_DEC_BM = 512


def _dec_body(h_ref, w_ref, b_ref, out_ref):
    d = jnp.dot(h_ref[...], w_ref[...], preferred_element_type=jnp.float32)
    d = d + b_ref[...]
    mx = jnp.max(d, axis=1, keepdims=True)
    z = d - mx
    s = jnp.sum(jnp.exp(z), axis=1, keepdims=True)
    out_ref[...] = (z - jnp.log(s))[:, :V]


def _decoder(hs2, w_p, b_p):
    return pl.pallas_call(
        _dec_body,
        grid=(N // _DEC_BM,),
        in_specs=[
            pl.BlockSpec((_DEC_BM, H), lambda i: (i, 0)),
            pl.BlockSpec((H, VP), lambda i: (0, 0)),
            pl.BlockSpec((1, VP), lambda i: (0, 0)),
        ],
        out_specs=pl.BlockSpec((_DEC_BM, V), lambda i: (i, 0)),
        out_shape=jax.ShapeDtypeStruct((N, V), jnp.float32),
    )(hs2, w_p, b_p)


def kernel(x, emb_table, W_ih, W_hh, b_ih, b_hh, W_out, b_out):
    wih_cat = jnp.concatenate([W_ih[0].T, W_ih[1].T], axis=1)
    bias2 = b_ih + b_hh
    emb_p = jnp.pad(emb_table, ((0, VP - V), (0, 0)))

    proj = _proj_table(emb_p, wih_cat, bias2)

    idx = x.T.reshape(-1).astype(jnp.int32)
    ih = _sc_gather(proj, idx)

    whh_cat = jnp.concatenate([W_hh[0].T, W_hh[1].T], axis=1)
    cellf = (x % 2).astype(jnp.float32)
    hs = _recurrence(cellf, ih.reshape(T, B, G),
                     whh_cat.astype(jnp.bfloat16))

    hs2 = hs.transpose(1, 0, 2).reshape(N, H)
    w_p = jnp.pad(W_out, ((0, VP - V), (0, 0))).T.astype(jnp.bfloat16)
    b_p = jnp.pad(b_out, (0, VP - V), constant_values=-1e30).reshape(1, VP)
    return _decoder(hs2, w_p, b_p)


# speedup vs baseline: 6.1101x; 1.0103x over previous
import functools

import jax
import jax.numpy as jnp
from jax import lax
from jax.experimental import pallas as pl
from jax.experimental.pallas import tpu as pltpu
import jax.experimental.pallas.tpu_sc as plsc

V = 1000
VP = 1024
E = 256
H = 1024
G = 4 * H
C = 2
B = 4
T = 2048
N = B * T


_PROJ_BM = 128


def _proj_body(emb_ref, w_ref, b_ref, out_ref):
    p = jnp.dot(emb_ref[...], w_ref[...], preferred_element_type=jnp.float32)
    par = lax.broadcasted_iota(jnp.int32, (_PROJ_BM, 1), 0) % 2
    even = p[:, :G] + b_ref[0:1, :]
    odd = p[:, G:] + b_ref[1:2, :]
    out_ref[...] = jnp.where(par == 0, even, odd)


def _proj_table(emb_p, wih_cat, bias2):
    return pl.pallas_call(
        _proj_body,
        grid=(VP // _PROJ_BM,),
        in_specs=[
            pl.BlockSpec((_PROJ_BM, E), lambda i: (i, 0)),
            pl.BlockSpec((E, C * G), lambda i: (0, 0)),
            pl.BlockSpec((C, G), lambda i: (0, 0)),
        ],
        out_specs=pl.BlockSpec((_PROJ_BM, G), lambda i: (i, 0)),
        out_shape=jax.ShapeDtypeStruct((VP, G), jnp.float32),
    )(emb_p, wih_cat, bias2)


_NW = 32
_ROWS_PER_W = N // _NW
_CH = 16


def _sc_gather(table, idx):
    mesh = plsc.VectorSubcoreMesh(core_axis_name="c", subcore_axis_name="s")

    @functools.partial(
        pl.kernel,
        out_type=jax.ShapeDtypeStruct((N, G), jnp.float32),
        mesh=mesh,
        scratch_types=[
            pltpu.VMEM((_CH,), jnp.int32),
            pltpu.VMEM((_CH, G), jnp.float32),
            pltpu.SemaphoreType.DMA,
        ],
    )
    def gather_k(table_hbm, idx_hbm, out_hbm, idx_v, rows_v, sem):
        wid = lax.axis_index("s") * 2 + lax.axis_index("c")
        base = wid * _ROWS_PER_W
        for ci in range(_ROWS_PER_W // _CH):
            start = base + ci * _CH
            pltpu.sync_copy(idx_hbm.at[pl.ds(start, _CH)], idx_v)
            pltpu.async_copy(table_hbm.at[idx_v], rows_v, sem).wait()
            pltpu.sync_copy(rows_v, out_hbm.at[pl.ds(start, _CH)])

    return gather_k(table, idx)


_TCH = 64


def _rec_body(cell_sm, ih_ref, whh_ref, hs_ref, h_ref, c_ref):
    blk = pl.program_id(0)

    @pl.when(blk == 0)
    def _():
        h_ref[...] = jnp.zeros((B, H), jnp.float32)
        c_ref[...] = jnp.zeros((B, H), jnp.float32)

    def one_step(tt, h, c):
        t = blk * _TCH + tt
        ih = ih_ref[tt]
        hh = jnp.dot(h.astype(jnp.bfloat16), whh_ref[...],
                     preferred_element_type=jnp.float32)
        rows = []
        for b in range(B):
            m = cell_sm[b, t]
            rows.append(ih[b:b + 1, :] + hh[b:b + 1, :G]
                        + m * (hh[b:b + 1, G:] - hh[b:b + 1, :G]))
        g = jnp.concatenate(rows, axis=0)
        i = jax.nn.sigmoid(g[:, 0 * H:1 * H])
        f = jax.nn.sigmoid(g[:, 1 * H:2 * H])
        gg = jnp.tanh(g[:, 2 * H:3 * H])
        o = jax.nn.sigmoid(g[:, 3 * H:4 * H])
        c2 = f * c + i * gg
        h2 = o * jnp.tanh(c2)
        hs_ref[tt] = h2.astype(jnp.bfloat16)
        return h2, c2

    def step(u, carry):
        h, c = carry
        for k in range(4):
            h, c = one_step(4 * u + k, h, c)
        return (h, c)

    h, c = lax.fori_loop(0, _TCH // 4, step, (h_ref[...], c_ref[...]))
    h_ref[...] = h
    c_ref[...] = c


def _recurrence(cellf, ih3, whh_b):
    return pl.pallas_call(
        _rec_body,
        grid=(T // _TCH,),
        in_specs=[
            pl.BlockSpec(memory_space=pltpu.SMEM),
            pl.BlockSpec((_TCH, B, G), lambda i: (i, 0, 0)),
            pl.BlockSpec((H, C * G), lambda i: (0, 0)),
        ],
        out_specs=pl.BlockSpec((_TCH, B, H), lambda i: (i, 0, 0)),
        out_shape=jax.ShapeDtypeStruct((T, B, H), jnp.bfloat16),
        scratch_shapes=[
            pltpu.VMEM((B, H), jnp.float32),
            pltpu.VMEM((B, H), jnp.float32),
        ],
        compiler_params=pltpu.CompilerParams(
            vmem_limit_bytes=112 * 1024 * 1024,
        ),
    )(cellf, ih3, whh_b)


_DEC_BM = 512


def _dec_body(h_ref, w_ref, b_ref, out_ref):
    d = jnp.dot(h_ref[...], w_ref[...], preferred_element_type=jnp.float32)
    d = d + b_ref[...]
    mx = jnp.max(d, axis=1, keepdims=True)
    z = d - mx
    s = jnp.sum(jnp.exp(z), axis=1, keepdims=True)
    out_ref[...] = (z - jnp.log(s))[:, :V]


def _decoder(hs2, w_p, b_p):
    return pl.pallas_call(
        _dec_body,
        grid=(N // _DEC_BM,),
        in_specs=[
            pl.BlockSpec((_DEC_BM, H), lambda i: (i, 0)),
            pl.BlockSpec((H, VP), lambda i: (0, 0)),
            pl.BlockSpec((1, VP), lambda i: (0, 0)),
        ],
        out_specs=pl.BlockSpec((_DEC_BM, V), lambda i: (i, 0)),
        out_shape=jax.ShapeDtypeStruct((N, V), jnp.float32),
    )(hs2, w_p, b_p)


def kernel(x, emb_table, W_ih, W_hh, b_ih, b_hh, W_out, b_out):
    wih_cat = jnp.concatenate([W_ih[0].T, W_ih[1].T], axis=1)
    bias2 = b_ih + b_hh
    emb_p = jnp.pad(emb_table, ((0, VP - V), (0, 0)))

    proj = _proj_table(emb_p, wih_cat, bias2)

    idx = x.T.reshape(-1).astype(jnp.int32)
    ih = _sc_gather(proj, idx)

    whh_cat = jnp.concatenate([W_hh[0].T, W_hh[1].T], axis=1)
    cellf = (x % 2).astype(jnp.float32)
    hs = _recurrence(cellf, ih.reshape(T, B, G),
                     whh_cat.astype(jnp.bfloat16))

    hs2 = hs.transpose(1, 0, 2).reshape(N, H)
    w_p = jnp.pad(W_out, ((0, VP - V), (0, 0))).T.astype(jnp.bfloat16)
    b_p = jnp.pad(b_out, (0, VP - V), constant_values=-1e30).reshape(1, VP)
    return _decoder(hs2, w_p, b_p)


# speedup vs baseline: 6.1459x; 1.0059x over previous
import functools

import jax
import jax.numpy as jnp
from jax import lax
from jax.experimental import pallas as pl
from jax.experimental.pallas import tpu as pltpu
import jax.experimental.pallas.tpu_sc as plsc

V = 1000
VP = 1024
E = 256
H = 1024
G = 4 * H
C = 2
B = 4
T = 2048
N = B * T


_PROJ_BM = 128


def _proj_body(emb_ref, w_ref, b_ref, out_ref):
    p = jnp.dot(emb_ref[...], w_ref[...], preferred_element_type=jnp.float32)
    par = lax.broadcasted_iota(jnp.int32, (_PROJ_BM, 1), 0) % 2
    even = p[:, :G] + b_ref[0:1, :]
    odd = p[:, G:] + b_ref[1:2, :]
    out_ref[...] = jnp.where(par == 0, even, odd)


def _proj_table(emb_p, wih_cat, bias2):
    return pl.pallas_call(
        _proj_body,
        grid=(VP // _PROJ_BM,),
        in_specs=[
            pl.BlockSpec((_PROJ_BM, E), lambda i: (i, 0)),
            pl.BlockSpec((E, C * G), lambda i: (0, 0)),
            pl.BlockSpec((C, G), lambda i: (0, 0)),
        ],
        out_specs=pl.BlockSpec((_PROJ_BM, G), lambda i: (i, 0)),
        out_shape=jax.ShapeDtypeStruct((VP, G), jnp.float32),
    )(emb_p, wih_cat, bias2)


_NW = 32
_ROWS_PER_W = N // _NW
_CH = 16


def _sc_gather(table, idx):
    mesh = plsc.VectorSubcoreMesh(core_axis_name="c", subcore_axis_name="s")

    @functools.partial(
        pl.kernel,
        out_type=jax.ShapeDtypeStruct((N, G), jnp.float32),
        mesh=mesh,
        scratch_types=[
            pltpu.VMEM((_CH,), jnp.int32),
            pltpu.VMEM((_CH, G), jnp.float32),
            pltpu.SemaphoreType.DMA,
        ],
    )
    def gather_k(table_hbm, idx_hbm, out_hbm, idx_v, rows_v, sem):
        wid = lax.axis_index("s") * 2 + lax.axis_index("c")
        base = wid * _ROWS_PER_W
        for ci in range(_ROWS_PER_W // _CH):
            start = base + ci * _CH
            pltpu.sync_copy(idx_hbm.at[pl.ds(start, _CH)], idx_v)
            pltpu.async_copy(table_hbm.at[idx_v], rows_v, sem).wait()
            pltpu.sync_copy(rows_v, out_hbm.at[pl.ds(start, _CH)])

    return gather_k(table, idx)


_TCH = 64


def _rec_body(cm_ref, ih_ref, whh_ref, hs_ref, h_ref, c_ref):
    blk = pl.program_id(0)

    @pl.when(blk == 0)
    def _():
        h_ref[...] = jnp.zeros((B, H), jnp.float32)
        c_ref[...] = jnp.zeros((B, H), jnp.float32)

    def one_step(tt, h, c):
        ih = ih_ref[tt]
        hh = jnp.dot(h.astype(jnp.bfloat16), whh_ref[...],
                     preferred_element_type=jnp.float32)
        m = cm_ref[tt]
        hh0 = hh[:, :G]
        g = ih + hh0 + m * (hh[:, G:] - hh0)
        i = jax.nn.sigmoid(g[:, 0 * H:1 * H])
        f = jax.nn.sigmoid(g[:, 1 * H:2 * H])
        gg = jnp.tanh(g[:, 2 * H:3 * H])
        o = jax.nn.sigmoid(g[:, 3 * H:4 * H])
        c2 = f * c + i * gg
        h2 = o * jnp.tanh(c2)
        hs_ref[tt] = h2.astype(jnp.bfloat16)
        return h2, c2

    def step(u, carry):
        h, c = carry
        for k in range(4):
            h, c = one_step(4 * u + k, h, c)
        return (h, c)

    h, c = lax.fori_loop(0, _TCH // 4, step, (h_ref[...], c_ref[...]))
    h_ref[...] = h
    c_ref[...] = c


def _recurrence(cellf, ih3, whh_b):
    return pl.pallas_call(
        _rec_body,
        grid=(T // _TCH,),
        in_specs=[
            pl.BlockSpec((_TCH, B, 1), lambda i: (i, 0, 0)),
            pl.BlockSpec((_TCH, B, G), lambda i: (i, 0, 0)),
            pl.BlockSpec((H, C * G), lambda i: (0, 0)),
        ],
        out_specs=pl.BlockSpec((_TCH, B, H), lambda i: (i, 0, 0)),
        out_shape=jax.ShapeDtypeStruct((T, B, H), jnp.bfloat16),
        scratch_shapes=[
            pltpu.VMEM((B, H), jnp.float32),
            pltpu.VMEM((B, H), jnp.float32),
        ],
        compiler_params=pltpu.CompilerParams(
            vmem_limit_bytes=112 * 1024 * 1024,
        ),
    )(cellf, ih3, whh_b)


_DEC_BM = 512


def _dec_body(h_ref, w_ref, b_ref, out_ref):
    d = jnp.dot(h_ref[...], w_ref[...], preferred_element_type=jnp.float32)
    d = d + b_ref[...]
    mx = jnp.max(d, axis=1, keepdims=True)
    z = d - mx
    s = jnp.sum(jnp.exp(z), axis=1, keepdims=True)
    out_ref[...] = (z - jnp.log(s))[:, :V]


def _decoder(hs2, w_p, b_p):
    return pl.pallas_call(
        _dec_body,
        grid=(N // _DEC_BM,),
        in_specs=[
            pl.BlockSpec((_DEC_BM, H), lambda i: (i, 0)),
            pl.BlockSpec((H, VP), lambda i: (0, 0)),
            pl.BlockSpec((1, VP), lambda i: (0, 0)),
        ],
        out_specs=pl.BlockSpec((_DEC_BM, V), lambda i: (i, 0)),
        out_shape=jax.ShapeDtypeStruct((N, V), jnp.float32),
    )(hs2, w_p, b_p)


def kernel(x, emb_table, W_ih, W_hh, b_ih, b_hh, W_out, b_out):
    wih_cat = jnp.concatenate([W_ih[0].T, W_ih[1].T], axis=1)
    bias2 = b_ih + b_hh
    emb_p = jnp.pad(emb_table, ((0, VP - V), (0, 0)))

    proj = _proj_table(emb_p, wih_cat, bias2)

    idx = x.T.reshape(-1).astype(jnp.int32)
    ih = _sc_gather(proj, idx)

    whh_cat = jnp.concatenate([W_hh[0].T, W_hh[1].T], axis=1)
    cellm = (x % 2).astype(jnp.float32).T.reshape(T, B, 1)
    hs = _recurrence(cellm, ih.reshape(T, B, G),
                     whh_cat.astype(jnp.bfloat16))

    hs2 = hs.transpose(1, 0, 2).reshape(N, H)
    w_p = jnp.pad(W_out, ((0, VP - V), (0, 0))).T.astype(jnp.bfloat16)
    b_p = jnp.pad(b_out, (0, VP - V), constant_values=-1e30).reshape(1, VP)
    return _decoder(hs2, w_p, b_p)


# speedup vs baseline: 6.1805x; 1.0056x over previous
import functools

import jax
import jax.numpy as jnp
from jax import lax
from jax.experimental import pallas as pl
from jax.experimental.pallas import tpu as pltpu
import jax.experimental.pallas.tpu_sc as plsc

V = 1000
VP = 1024
E = 256
H = 1024
G = 4 * H
C = 2
B = 4
T = 2048
N = B * T


_PROJ_BM = 128


def _proj_body(emb_ref, w_ref, b_ref, out_ref):
    p = jnp.dot(emb_ref[...], w_ref[...], preferred_element_type=jnp.float32)
    par = lax.broadcasted_iota(jnp.int32, (_PROJ_BM, 1), 0) % 2
    even = p[:, :G] + b_ref[0:1, :]
    odd = p[:, G:] + b_ref[1:2, :]
    out_ref[...] = jnp.where(par == 0, even, odd)


def _proj_table(emb_p, wih_cat, bias2):
    return pl.pallas_call(
        _proj_body,
        grid=(VP // _PROJ_BM,),
        in_specs=[
            pl.BlockSpec((_PROJ_BM, E), lambda i: (i, 0)),
            pl.BlockSpec((E, C * G), lambda i: (0, 0)),
            pl.BlockSpec((C, G), lambda i: (0, 0)),
        ],
        out_specs=pl.BlockSpec((_PROJ_BM, G), lambda i: (i, 0)),
        out_shape=jax.ShapeDtypeStruct((VP, G), jnp.float32),
    )(emb_p, wih_cat, bias2)


_NW = 32
_ROWS_PER_W = N // _NW
_CH = 16


def _sc_gather(table, idx):
    mesh = plsc.VectorSubcoreMesh(core_axis_name="c", subcore_axis_name="s")

    @functools.partial(
        pl.kernel,
        out_type=jax.ShapeDtypeStruct((N, G), jnp.float32),
        mesh=mesh,
        scratch_types=[
            pltpu.VMEM((_CH,), jnp.int32),
            pltpu.VMEM((_CH, G), jnp.float32),
            pltpu.SemaphoreType.DMA,
        ],
    )
    def gather_k(table_hbm, idx_hbm, out_hbm, idx_v, rows_v, sem):
        wid = lax.axis_index("s") * 2 + lax.axis_index("c")
        base = wid * _ROWS_PER_W
        for ci in range(_ROWS_PER_W // _CH):
            start = base + ci * _CH
            pltpu.sync_copy(idx_hbm.at[pl.ds(start, _CH)], idx_v)
            pltpu.async_copy(table_hbm.at[idx_v], rows_v, sem).wait()
            pltpu.sync_copy(rows_v, out_hbm.at[pl.ds(start, _CH)])

    return gather_k(table, idx)


_TCH = 128


def _rec_body(cm_ref, ih_ref, whh_ref, hs_ref, h_ref, c_ref):
    blk = pl.program_id(0)

    @pl.when(blk == 0)
    def _():
        h_ref[...] = jnp.zeros((B, H), jnp.float32)
        c_ref[...] = jnp.zeros((B, H), jnp.float32)

    def one_step(tt, h, c):
        ih = ih_ref[tt]
        hh = jnp.dot(h.astype(jnp.bfloat16), whh_ref[...],
                     preferred_element_type=jnp.float32)
        m = cm_ref[tt]
        hh0 = hh[:, :G]
        g = ih + hh0 + m * (hh[:, G:] - hh0)
        i = jax.nn.sigmoid(g[:, 0 * H:1 * H])
        f = jax.nn.sigmoid(g[:, 1 * H:2 * H])
        gg = jnp.tanh(g[:, 2 * H:3 * H])
        o = jax.nn.sigmoid(g[:, 3 * H:4 * H])
        c2 = f * c + i * gg
        h2 = o * jnp.tanh(c2)
        hs_ref[tt] = h2.astype(jnp.bfloat16)
        return h2, c2

    def step(u, carry):
        h, c = carry
        for k in range(8):
            h, c = one_step(8 * u + k, h, c)
        return (h, c)

    h, c = lax.fori_loop(0, _TCH // 8, step, (h_ref[...], c_ref[...]))
    h_ref[...] = h
    c_ref[...] = c


def _recurrence(cellf, ih3, whh_b):
    return pl.pallas_call(
        _rec_body,
        grid=(T // _TCH,),
        in_specs=[
            pl.BlockSpec((_TCH, B, 1), lambda i: (i, 0, 0)),
            pl.BlockSpec((_TCH, B, G), lambda i: (i, 0, 0)),
            pl.BlockSpec((H, C * G), lambda i: (0, 0)),
        ],
        out_specs=pl.BlockSpec((_TCH, B, H), lambda i: (i, 0, 0)),
        out_shape=jax.ShapeDtypeStruct((T, B, H), jnp.bfloat16),
        scratch_shapes=[
            pltpu.VMEM((B, H), jnp.float32),
            pltpu.VMEM((B, H), jnp.float32),
        ],
        compiler_params=pltpu.CompilerParams(
            vmem_limit_bytes=112 * 1024 * 1024,
        ),
    )(cellf, ih3, whh_b)


_DEC_BM = 512


def _dec_body(h_ref, w_ref, b_ref, out_ref):
    d = jnp.dot(h_ref[...], w_ref[...], preferred_element_type=jnp.float32)
    d = d + b_ref[...]
    mx = jnp.max(d, axis=1, keepdims=True)
    z = d - mx
    s = jnp.sum(jnp.exp(z), axis=1, keepdims=True)
    out_ref[...] = (z - jnp.log(s))[:, :V]


def _decoder(hs2, w_p, b_p):
    return pl.pallas_call(
        _dec_body,
        grid=(N // _DEC_BM,),
        in_specs=[
            pl.BlockSpec((_DEC_BM, H), lambda i: (i, 0)),
            pl.BlockSpec((H, VP), lambda i: (0, 0)),
            pl.BlockSpec((1, VP), lambda i: (0, 0)),
        ],
        out_specs=pl.BlockSpec((_DEC_BM, V), lambda i: (i, 0)),
        out_shape=jax.ShapeDtypeStruct((N, V), jnp.float32),
    )(hs2, w_p, b_p)


def kernel(x, emb_table, W_ih, W_hh, b_ih, b_hh, W_out, b_out):
    wih_cat = jnp.concatenate([W_ih[0].T, W_ih[1].T], axis=1)
    bias2 = b_ih + b_hh
    emb_p = jnp.pad(emb_table, ((0, VP - V), (0, 0)))

    proj = _proj_table(emb_p, wih_cat, bias2)

    idx = x.T.reshape(-1).astype(jnp.int32)
    ih = _sc_gather(proj, idx)

    whh_cat = jnp.concatenate([W_hh[0].T, W_hh[1].T], axis=1)
    cellm = (x % 2).astype(jnp.float32).T.reshape(T, B, 1)
    hs = _recurrence(cellm, ih.reshape(T, B, G),
                     whh_cat.astype(jnp.bfloat16))

    hs2 = hs.transpose(1, 0, 2).reshape(N, H)
    w_p = jnp.pad(W_out, ((0, VP - V), (0, 0))).T.astype(jnp.bfloat16)
    b_p = jnp.pad(b_out, (0, VP - V), constant_values=-1e30).reshape(1, VP)
    return _decoder(hs2, w_p, b_p)


# speedup vs baseline: 6.1929x; 1.0020x over previous
import functools

import jax
import jax.numpy as jnp
from jax import lax
from jax.experimental import pallas as pl
from jax.experimental.pallas import tpu as pltpu
import jax.experimental.pallas.tpu_sc as plsc

V = 1000
VP = 1024
E = 256
H = 1024
G = 4 * H
C = 2
B = 4
T = 2048
N = B * T


_PROJ_BM = 128


def _proj_body(emb_ref, w_ref, b_ref, out_ref):
    p = jnp.dot(emb_ref[...], w_ref[...], preferred_element_type=jnp.float32)
    par = lax.broadcasted_iota(jnp.int32, (_PROJ_BM, 1), 0) % 2
    even = p[:, :G] + b_ref[0:1, :]
    odd = p[:, G:] + b_ref[1:2, :]
    out_ref[...] = jnp.where(par == 0, even, odd)


def _proj_table(emb_p, wih_cat, bias2):
    return pl.pallas_call(
        _proj_body,
        grid=(VP // _PROJ_BM,),
        in_specs=[
            pl.BlockSpec((_PROJ_BM, E), lambda i: (i, 0)),
            pl.BlockSpec((E, C * G), lambda i: (0, 0)),
            pl.BlockSpec((C, G), lambda i: (0, 0)),
        ],
        out_specs=pl.BlockSpec((_PROJ_BM, G), lambda i: (i, 0)),
        out_shape=jax.ShapeDtypeStruct((VP, G), jnp.float32),
    )(emb_p, wih_cat, bias2)


_NW = 32
_ROWS_PER_W = N // _NW
_CH = 8
_NCH = _ROWS_PER_W // _CH


def _sc_gather(table, idx):
    mesh = plsc.VectorSubcoreMesh(core_axis_name="c", subcore_axis_name="s")

    @functools.partial(
        pl.kernel,
        out_type=jax.ShapeDtypeStruct((N, G), jnp.float32),
        mesh=mesh,
        scratch_types=[
            pltpu.VMEM((_ROWS_PER_W,), jnp.int32),
            pltpu.VMEM((_CH, G), jnp.float32),
            pltpu.VMEM((_CH, G), jnp.float32),
            pltpu.SemaphoreType.DMA,
            pltpu.SemaphoreType.DMA,
        ],
    )
    def gather_k(table_hbm, idx_hbm, out_hbm, idx_v, rows0, rows1, sem0, sem1):
        wid = lax.axis_index("s") * 2 + lax.axis_index("c")
        base = wid * _ROWS_PER_W
        pltpu.sync_copy(idx_hbm.at[pl.ds(base, _ROWS_PER_W)], idx_v)
        bufs = (rows0, rows1)
        sems = (sem0, sem1)

        def start(k):
            pltpu.make_async_copy(
                table_hbm.at[idx_v.at[pl.ds(k * _CH, _CH)]],
                bufs[k % 2], sems[k % 2]).start()

        start(0)
        for k in range(_NCH):
            if k + 1 < _NCH:
                start(k + 1)
            pltpu.make_async_copy(
                table_hbm.at[idx_v.at[pl.ds(k * _CH, _CH)]],
                bufs[k % 2], sems[k % 2]).wait()
            pltpu.sync_copy(bufs[k % 2],
                            out_hbm.at[pl.ds(base + k * _CH, _CH)])

    return gather_k(table, idx)


_TCH = 128


def _rec_body(cm_ref, ih_ref, whh_ref, hs_ref, h_ref, c_ref):
    blk = pl.program_id(0)

    @pl.when(blk == 0)
    def _():
        h_ref[...] = jnp.zeros((B, H), jnp.float32)
        c_ref[...] = jnp.zeros((B, H), jnp.float32)

    def one_step(tt, h, c):
        ih = ih_ref[tt]
        hh = jnp.dot(h.astype(jnp.bfloat16), whh_ref[...],
                     preferred_element_type=jnp.float32)
        m = cm_ref[tt]
        hh0 = hh[:, :G]
        g = ih + hh0 + m * (hh[:, G:] - hh0)
        i = jax.nn.sigmoid(g[:, 0 * H:1 * H])
        f = jax.nn.sigmoid(g[:, 1 * H:2 * H])
        gg = jnp.tanh(g[:, 2 * H:3 * H])
        o = jax.nn.sigmoid(g[:, 3 * H:4 * H])
        c2 = f * c + i * gg
        h2 = o * jnp.tanh(c2)
        hs_ref[tt] = h2.astype(jnp.bfloat16)
        return h2, c2

    def step(u, carry):
        h, c = carry
        for k in range(8):
            h, c = one_step(8 * u + k, h, c)
        return (h, c)

    h, c = lax.fori_loop(0, _TCH // 8, step, (h_ref[...], c_ref[...]))
    h_ref[...] = h
    c_ref[...] = c


def _recurrence(cellf, ih3, whh_b):
    return pl.pallas_call(
        _rec_body,
        grid=(T // _TCH,),
        in_specs=[
            pl.BlockSpec((_TCH, B, 1), lambda i: (i, 0, 0)),
            pl.BlockSpec((_TCH, B, G), lambda i: (i, 0, 0)),
            pl.BlockSpec((H, C * G), lambda i: (0, 0)),
        ],
        out_specs=pl.BlockSpec((_TCH, B, H), lambda i: (i, 0, 0)),
        out_shape=jax.ShapeDtypeStruct((T, B, H), jnp.bfloat16),
        scratch_shapes=[
            pltpu.VMEM((B, H), jnp.float32),
            pltpu.VMEM((B, H), jnp.float32),
        ],
        compiler_params=pltpu.CompilerParams(
            vmem_limit_bytes=112 * 1024 * 1024,
        ),
    )(cellf, ih3, whh_b)


_DEC_BM = 512


def _dec_body(h_ref, w_ref, b_ref, out_ref):
    d = jnp.dot(h_ref[...], w_ref[...], preferred_element_type=jnp.float32)
    d = d + b_ref[...]
    mx = jnp.max(d, axis=1, keepdims=True)
    z = d - mx
    s = jnp.sum(jnp.exp(z), axis=1, keepdims=True)
    out_ref[...] = (z - jnp.log(s))[:, :V]


def _decoder(hs2, w_p, b_p):
    return pl.pallas_call(
        _dec_body,
        grid=(N // _DEC_BM,),
        in_specs=[
            pl.BlockSpec((_DEC_BM, H), lambda i: (i, 0)),
            pl.BlockSpec((H, VP), lambda i: (0, 0)),
            pl.BlockSpec((1, VP), lambda i: (0, 0)),
        ],
        out_specs=pl.BlockSpec((_DEC_BM, V), lambda i: (i, 0)),
        out_shape=jax.ShapeDtypeStruct((N, V), jnp.float32),
    )(hs2, w_p, b_p)


def kernel(x, emb_table, W_ih, W_hh, b_ih, b_hh, W_out, b_out):
    wih_cat = jnp.concatenate([W_ih[0].T, W_ih[1].T], axis=1)
    bias2 = b_ih + b_hh
    emb_p = jnp.pad(emb_table, ((0, VP - V), (0, 0)))

    proj = _proj_table(emb_p, wih_cat, bias2)

    idx = x.T.reshape(-1).astype(jnp.int32)
    ih = _sc_gather(proj, idx)

    whh_cat = jnp.concatenate([W_hh[0].T, W_hh[1].T], axis=1)
    cellm = (x % 2).astype(jnp.float32).T.reshape(T, B, 1)
    hs = _recurrence(cellm, ih.reshape(T, B, G),
                     whh_cat.astype(jnp.bfloat16))

    hs2 = hs.transpose(1, 0, 2).reshape(N, H)
    w_p = jnp.pad(W_out, ((0, VP - V), (0, 0))).T.astype(jnp.bfloat16)
    b_p = jnp.pad(b_out, (0, VP - V), constant_values=-1e30).reshape(1, VP)
    return _decoder(hs2, w_p, b_p)


# speedup vs baseline: 6.1956x; 1.0004x over previous
import functools

import jax
import jax.numpy as jnp
from jax import lax
from jax.experimental import pallas as pl
from jax.experimental.pallas import tpu as pltpu
import jax.experimental.pallas.tpu_sc as plsc

V = 1000
VP = 1024
E = 256
H = 1024
G = 4 * H
C = 2
B = 4
T = 2048
N = B * T


_PROJ_BM = 128


def _proj_body(emb_ref, w_ref, b_ref, out_ref):
    p = jnp.dot(emb_ref[...], w_ref[...], preferred_element_type=jnp.float32)
    par = lax.broadcasted_iota(jnp.int32, (_PROJ_BM, 1), 0) % 2
    even = p[:, :G] + b_ref[0:1, :]
    odd = p[:, G:] + b_ref[1:2, :]
    out_ref[...] = jnp.where(par == 0, even, odd)


def _proj_table(emb_p, wih_cat, bias2):
    return pl.pallas_call(
        _proj_body,
        grid=(VP // _PROJ_BM,),
        in_specs=[
            pl.BlockSpec((_PROJ_BM, E), lambda i: (i, 0)),
            pl.BlockSpec((E, C * G), lambda i: (0, 0)),
            pl.BlockSpec((C, G), lambda i: (0, 0)),
        ],
        out_specs=pl.BlockSpec((_PROJ_BM, G), lambda i: (i, 0)),
        out_shape=jax.ShapeDtypeStruct((VP, G), jnp.float32),
    )(emb_p, wih_cat, bias2)


_NW = 32
_ROWS_PER_W = N // _NW
_CH = 8
_NCH = _ROWS_PER_W // _CH


def _sc_gather(table, idx):
    mesh = plsc.VectorSubcoreMesh(core_axis_name="c", subcore_axis_name="s")

    @functools.partial(
        pl.kernel,
        out_type=jax.ShapeDtypeStruct((N, G), jnp.float32),
        mesh=mesh,
        scratch_types=[
            pltpu.VMEM((_ROWS_PER_W,), jnp.int32),
            pltpu.VMEM((_CH, G), jnp.float32),
            pltpu.VMEM((_CH, G), jnp.float32),
            pltpu.SemaphoreType.DMA,
            pltpu.SemaphoreType.DMA,
        ],
    )
    def gather_k(table_hbm, idx_hbm, out_hbm, idx_v, rows0, rows1, sem0, sem1):
        wid = lax.axis_index("s") * 2 + lax.axis_index("c")
        base = wid * _ROWS_PER_W
        pltpu.sync_copy(idx_hbm.at[pl.ds(base, _ROWS_PER_W)], idx_v)
        bufs = (rows0, rows1)
        sems = (sem0, sem1)

        def start(k):
            pltpu.make_async_copy(
                table_hbm.at[idx_v.at[pl.ds(k * _CH, _CH)]],
                bufs[k % 2], sems[k % 2]).start()

        start(0)
        for k in range(_NCH):
            if k + 1 < _NCH:
                start(k + 1)
            pltpu.make_async_copy(
                table_hbm.at[idx_v.at[pl.ds(k * _CH, _CH)]],
                bufs[k % 2], sems[k % 2]).wait()
            pltpu.sync_copy(bufs[k % 2],
                            out_hbm.at[pl.ds(base + k * _CH, _CH)])

    return gather_k(table, idx)


_TCH = 128


def _rec_body(cm_ref, ih_ref, whh_ref, hs_ref, h_ref, c_ref):
    blk = pl.program_id(0)

    @pl.when(blk == 0)
    def _():
        h_ref[...] = jnp.zeros((B, H), jnp.float32)
        c_ref[...] = jnp.zeros((B, H), jnp.float32)

    def one_step(tt, h, c):
        ih = ih_ref[tt]
        hh = jnp.dot(h.astype(jnp.bfloat16), whh_ref[...],
                     preferred_element_type=jnp.float32)
        m = cm_ref[tt]
        g = ih + jnp.where(m > 0.5, hh[:, G:], hh[:, :G])
        i = jax.nn.sigmoid(g[:, 0 * H:1 * H])
        f = jax.nn.sigmoid(g[:, 1 * H:2 * H])
        gg = jnp.tanh(g[:, 2 * H:3 * H])
        o = jax.nn.sigmoid(g[:, 3 * H:4 * H])
        c2 = f * c + i * gg
        h2 = o * jnp.tanh(c2)
        hs_ref[tt] = h2.astype(jnp.bfloat16)
        return h2, c2

    def step(u, carry):
        h, c = carry
        for k in range(8):
            h, c = one_step(8 * u + k, h, c)
        return (h, c)

    h, c = lax.fori_loop(0, _TCH // 8, step, (h_ref[...], c_ref[...]))
    h_ref[...] = h
    c_ref[...] = c


def _recurrence(cellf, ih3, whh_b):
    return pl.pallas_call(
        _rec_body,
        grid=(T // _TCH,),
        in_specs=[
            pl.BlockSpec((_TCH, B, 1), lambda i: (i, 0, 0)),
            pl.BlockSpec((_TCH, B, G), lambda i: (i, 0, 0)),
            pl.BlockSpec((H, C * G), lambda i: (0, 0)),
        ],
        out_specs=pl.BlockSpec((_TCH, B, H), lambda i: (i, 0, 0)),
        out_shape=jax.ShapeDtypeStruct((T, B, H), jnp.bfloat16),
        scratch_shapes=[
            pltpu.VMEM((B, H), jnp.float32),
            pltpu.VMEM((B, H), jnp.float32),
        ],
        compiler_params=pltpu.CompilerParams(
            vmem_limit_bytes=112 * 1024 * 1024,
        ),
    )(cellf, ih3, whh_b)


_DEC_BM = 512


def _dec_body(h_ref, w_ref, b_ref, out_ref):
    d = jnp.dot(h_ref[...], w_ref[...], preferred_element_type=jnp.float32)
    d = d + b_ref[...]
    mx = jnp.max(d, axis=1, keepdims=True)
    z = d - mx
    s = jnp.sum(jnp.exp(z), axis=1, keepdims=True)
    out_ref[...] = (z - jnp.log(s))[:, :V]


def _decoder(hs2, w_p, b_p):
    return pl.pallas_call(
        _dec_body,
        grid=(N // _DEC_BM,),
        in_specs=[
            pl.BlockSpec((_DEC_BM, H), lambda i: (i, 0)),
            pl.BlockSpec((H, VP), lambda i: (0, 0)),
            pl.BlockSpec((1, VP), lambda i: (0, 0)),
        ],
        out_specs=pl.BlockSpec((_DEC_BM, V), lambda i: (i, 0)),
        out_shape=jax.ShapeDtypeStruct((N, V), jnp.float32),
    )(hs2, w_p, b_p)


def kernel(x, emb_table, W_ih, W_hh, b_ih, b_hh, W_out, b_out):
    wih_cat = jnp.concatenate([W_ih[0].T, W_ih[1].T], axis=1)
    bias2 = b_ih + b_hh
    emb_p = jnp.pad(emb_table, ((0, VP - V), (0, 0)))

    proj = _proj_table(emb_p, wih_cat, bias2)

    idx = x.T.reshape(-1).astype(jnp.int32)
    ih = _sc_gather(proj, idx)

    whh_cat = jnp.concatenate([W_hh[0].T, W_hh[1].T], axis=1)
    cellm = (x % 2).astype(jnp.float32).T.reshape(T, B, 1)
    hs = _recurrence(cellm, ih.reshape(T, B, G),
                     whh_cat.astype(jnp.bfloat16))

    hs2 = hs.transpose(1, 0, 2).reshape(N, H)
    w_p = jnp.pad(W_out, ((0, VP - V), (0, 0))).T.astype(jnp.bfloat16)
    b_p = jnp.pad(b_out, (0, VP - V), constant_values=-1e30).reshape(1, VP)
    return _decoder(hs2, w_p, b_p)
